# baseline (speedup 1.0000x reference)
import functools

import jax
import jax.numpy as jnp
from jax import lax
from jax.experimental import pallas as pl
from jax.experimental.pallas import tpu as pltpu
from jax.experimental.pallas import tpu_sc as plsc

_NC = 2
_NS = 16
_NW = _NC * _NS
_BLK = 128
_TB = 512
_S = 512
_TC_FRAC = 0.25


def _sc_body(feat_hbm, ids_hbm, out_hbm, ids_v, bufs, zbuf, acc,
             sem0, sem1, sem2, sem3,
             *, total_blocks, qsb, extra_sb, rem, nb, d):
    c = lax.axis_index("c")
    s = lax.axis_index("s")
    wid = s * _NC + c
    nblk = 8 * (qsb + (wid < extra_sb).astype(jnp.int32)) \
        + (wid == _NW - 1).astype(jnp.int32) * rem
    start = 8 * (wid * qsb + jnp.minimum(wid, extra_sb))

    def _blk_slice(i):
        bi = jnp.minimum(start + i, total_blocks - 1)
        return feat_hbm.at[pl.ds(bi * _BLK, _BLK)]

    sems = (sem0, sem1, sem2, sem3)
    pltpu.async_copy(_blk_slice(0), bufs.at[0], sems[0])
    pltpu.async_copy(_blk_slice(1), bufs.at[1], sems[1])
    pltpu.async_copy(_blk_slice(2), bufs.at[2], sems[2])

    zero = jnp.zeros((16,), jnp.float32)
    for r in range(_S // _NS):
        for k8 in range(d // 16):
            zbuf[r, pl.ds(k8 * 16, 16)] = zero
    pltpu.sync_copy(zbuf, acc.at[pl.ds(s * (_S // _NS), _S // _NS)])

    pltpu.sync_copy(ids_hbm.at[pl.ds(start, nb)], ids_v)

    plsc.subcore_barrier()

    def _outer(g, carry):
        for b in range(4):
            i = 4 * g + b
            pltpu.make_async_copy(_blk_slice(0), bufs.at[b], sems[b]).wait()

            @pl.when(i + 3 < nb)
            def _refill():
                b2 = (b + 3) % 4
                pltpu.async_copy(_blk_slice(i + 3), bufs.at[b2], sems[b2])

            @pl.when(i < nblk)
            def _scatter():
                pltpu.sync_copy(bufs.at[b], acc.at[ids_v.at[i]], add=True)
        return carry

    lax.fori_loop(0, nb // 4, _outer, 0)

    plsc.subcore_barrier()
    rows = _S // _NS
    pltpu.sync_copy(acc.at[pl.ds(s * rows, rows)],
                    out_hbm.at[pl.ds(c * _S + s * rows, rows)])


def _tc_body(ids_ref, feat_ref, out_ref):
    @pl.when(pl.program_id(0) == 0)
    def _init():
        out_ref[...] = jnp.zeros_like(out_ref)

    ids_blk = ids_ref[0, 0, :]
    oh_t = (lax.broadcasted_iota(jnp.int32, (_S, _TB), 0)
            == ids_blk[None, :]).astype(jnp.bfloat16)
    fb = feat_ref[...].astype(jnp.bfloat16)
    out_ref[...] += jnp.dot(oh_t, fb, preferred_element_type=jnp.float32)


def _combine_body(p_ref, t0_ref, t1_ref, o_ref):
    o_ref[...] = p_ref[:_S, :] + p_ref[_S:, :] + t0_ref[...] + t1_ref[...]


def kernel(feat, segment_ids, num_segments):
    e, d = feat.shape
    ntc = int(e // _TB * _TC_FRAC)
    e_sc = e - ntc * _TB

    total_blocks = e_sc // _BLK
    total_sb = total_blocks // 8
    rem = total_blocks - 8 * total_sb
    qsb = total_sb // _NW
    extra_sb = total_sb - qsb * _NW
    max_blk = max(8 * (qsb + (1 if extra_sb else 0)), 8 * qsb + rem)
    nb = ((max_blk + 3) // 4) * 4

    ids = (segment_ids + (num_segments - _S)).astype(jnp.int32)
    pad_rows = total_blocks + nb
    ids2d = jnp.pad(ids[:e_sc], (0, pad_rows * _BLK - e_sc)).reshape(pad_rows, _BLK)

    mesh = plsc.VectorSubcoreMesh(core_axis_name="c", subcore_axis_name="s",
                                  num_cores=_NC, num_subcores=_NS)
    body = functools.partial(_sc_body, total_blocks=total_blocks, qsb=qsb,
                             extra_sb=extra_sb, rem=rem, nb=nb, d=d)
    partials = pl.kernel(
        body,
        jax.ShapeDtypeStruct((_NC * _S, d), jnp.float32),
        mesh=mesh,
        scratch_types=[
            pltpu.VMEM((nb, _BLK), jnp.int32),
            pltpu.VMEM((4, _BLK, d), jnp.float32),
            pltpu.VMEM((_S // _NS, d), jnp.float32),
            pltpu.VMEM_SHARED((_S, d), jnp.float32),
            pltpu.SemaphoreType.DMA,
            pltpu.SemaphoreType.DMA,
            pltpu.SemaphoreType.DMA,
            pltpu.SemaphoreType.DMA,
        ],
    )(feat, ids2d)

    ids_tc = ids[e_sc:].reshape(ntc, 1, _TB)
    nh = ntc // 2
    tc_parts = []
    for piece, (lo, cnt) in enumerate([(0, nh), (nh, ntc - nh)]):
        tc_parts.append(pl.pallas_call(
            _tc_body,
            grid=(cnt,),
            in_specs=[
                pl.BlockSpec((1, 1, _TB), lambda g, lo=lo: (lo + g, 0, 0)),
                pl.BlockSpec((_TB, d),
                             lambda g, lo=lo: (e_sc // _TB + lo + g, 0)),
            ],
            out_specs=pl.BlockSpec((_S, d), lambda g: (0, 0)),
            out_shape=jax.ShapeDtypeStruct((_S, d), jnp.float32),
        )(ids_tc, feat))

    return pl.pallas_call(
        _combine_body,
        out_shape=jax.ShapeDtypeStruct((_S, d), jnp.float32),
    )(partials, tc_parts[0], tc_parts[1])


# speedup vs baseline: 1.2358x; 1.2358x over previous
import functools

import jax
import jax.numpy as jnp
from jax import lax
from jax.experimental import pallas as pl
from jax.experimental.pallas import tpu as pltpu
from jax.experimental.pallas import tpu_sc as plsc

_NC = 2
_NS = 16
_NW = _NC * _NS
_BLK = 128
_S = 512


def _sc_body(feat_hbm, ids_hbm, out_hbm, ids_raw, ids_v, bufs, zbuf, acc,
             sem0, sem1, sem2, sem3,
             *, total_blocks, qsb, extra_sb, rem, nb, d):
    c = lax.axis_index("c")
    s = lax.axis_index("s")
    wid = s * _NC + c
    nblk = 8 * (qsb + (wid < extra_sb).astype(jnp.int32)) \
        + (wid == _NW - 1).astype(jnp.int32) * rem
    start = 8 * (wid * qsb + jnp.minimum(wid, extra_sb))

    def _blk_slice(i):
        bi = jnp.minimum(start + i, total_blocks - 1)
        return feat_hbm.at[pl.ds(bi * _BLK, _BLK)]

    sems = (sem0, sem1, sem2, sem3)
    pltpu.async_copy(_blk_slice(0), bufs.at[0], sems[0])
    pltpu.async_copy(_blk_slice(1), bufs.at[1], sems[1])
    pltpu.async_copy(_blk_slice(2), bufs.at[2], sems[2])

    w = nb + 8
    wstart = jnp.minimum(start, total_blocks - w)
    shift = start - wstart
    pltpu.sync_copy(ids_hbm.at[pl.ds(wstart * _BLK, w * _BLK)], ids_raw)

    def _repack(r, carry):
        rr = jnp.minimum(r + shift, w - 1) * _BLK
        for k8 in range(_BLK // 16):
            ids_v[r, pl.ds(k8 * 16, 16)] = ids_raw[pl.ds(rr + k8 * 16, 16)]
        return carry

    lax.fori_loop(0, nb, _repack, 0)

    zero = jnp.zeros((16,), jnp.float32)
    for r in range(_S // _NS):
        for k8 in range(d // 16):
            zbuf[r, pl.ds(k8 * 16, 16)] = zero
    pltpu.sync_copy(zbuf, acc.at[pl.ds(s * (_S // _NS), _S // _NS)])

    plsc.subcore_barrier()

    def _outer(g, carry):
        for b in range(4):
            i = 4 * g + b
            pltpu.make_async_copy(_blk_slice(0), bufs.at[b], sems[b]).wait()

            @pl.when(i + 3 < nb)
            def _refill():
                b2 = (b + 3) % 4
                pltpu.async_copy(_blk_slice(i + 3), bufs.at[b2], sems[b2])

            @pl.when(i < nblk)
            def _scatter():
                pltpu.sync_copy(bufs.at[b], acc.at[ids_v.at[i]], add=True)
        return carry

    lax.fori_loop(0, nb // 4, _outer, 0)

    plsc.subcore_barrier()
    rows = _S // _NS
    pltpu.sync_copy(acc.at[pl.ds(s * rows, rows)],
                    out_hbm.at[pl.ds(c * _S + s * rows, rows)])


def _combine_body(p_ref, o_ref):
    o_ref[...] = p_ref[:_S, :] + p_ref[_S:, :]


def kernel(feat, segment_ids, num_segments):
    del num_segments
    e, d = feat.shape
    total_blocks = e // _BLK
    total_sb = total_blocks // 8
    rem = total_blocks - 8 * total_sb
    qsb = total_sb // _NW
    extra_sb = total_sb - qsb * _NW
    max_blk = max(8 * (qsb + (1 if extra_sb else 0)), 8 * qsb + rem)
    nb = ((max_blk + 3) // 4) * 4

    ids = segment_ids.astype(jnp.int32)

    mesh = plsc.VectorSubcoreMesh(core_axis_name="c", subcore_axis_name="s",
                                  num_cores=_NC, num_subcores=_NS)
    body = functools.partial(_sc_body, total_blocks=total_blocks, qsb=qsb,
                             extra_sb=extra_sb, rem=rem, nb=nb, d=d)
    partials = pl.kernel(
        body,
        jax.ShapeDtypeStruct((_NC * _S, d), jnp.float32),
        mesh=mesh,
        scratch_types=[
            pltpu.VMEM(((nb + 8) * _BLK,), jnp.int32),
            pltpu.VMEM((nb, _BLK), jnp.int32),
            pltpu.VMEM((4, _BLK, d), jnp.float32),
            pltpu.VMEM((_S // _NS, d), jnp.float32),
            pltpu.VMEM_SHARED((_S, d), jnp.float32),
            pltpu.SemaphoreType.DMA,
            pltpu.SemaphoreType.DMA,
            pltpu.SemaphoreType.DMA,
            pltpu.SemaphoreType.DMA,
        ],
    )(feat, ids)

    return pl.pallas_call(
        _combine_body,
        out_shape=jax.ShapeDtypeStruct((_S, d), jnp.float32),
    )(partials)


# speedup vs baseline: 1.2369x; 1.0009x over previous
import functools

import jax
import jax.numpy as jnp
from jax import lax
from jax.experimental import pallas as pl
from jax.experimental.pallas import tpu as pltpu
from jax.experimental.pallas import tpu_sc as plsc

_NC = 2
_NS = 16
_NW = _NC * _NS
_BLK = 128
_S = 512


def _sc_body(feat_hbm, ids_hbm, out_hbm, ids_raw, ids_v, bufs, zbuf, acc,
             sem0, sem1, sem2, sem3,
             *, total_blocks, qsb, extra_sb, rem, nb, d):
    c = lax.axis_index("c")
    s = lax.axis_index("s")
    wid = s * _NC + c
    nblk = 8 * (qsb + (wid < extra_sb).astype(jnp.int32)) \
        + (wid == _NW - 1).astype(jnp.int32) * rem
    start = 8 * (wid * qsb + jnp.minimum(wid, extra_sb))

    def _blk_slice(i):
        bi = jnp.minimum(start + i, total_blocks - 1)
        return feat_hbm.at[pl.ds(bi * _BLK, _BLK)]

    sems = (sem0, sem1, sem2, sem3)
    pltpu.async_copy(_blk_slice(0), bufs.at[0], sems[0])
    pltpu.async_copy(_blk_slice(1), bufs.at[1], sems[1])
    pltpu.async_copy(_blk_slice(2), bufs.at[2], sems[2])

    w = nb + 8
    wstart = jnp.minimum(start, total_blocks - w)
    shift = start - wstart
    pltpu.sync_copy(ids_hbm.at[pl.ds(wstart * _BLK, w * _BLK)], ids_raw)

    def _repack(r):
        rr = jnp.minimum(r + shift, w - 1) * _BLK
        for k8 in range(_BLK // 16):
            ids_v[r, pl.ds(k8 * 16, 16)] = ids_raw[pl.ds(rr + k8 * 16, 16)]

    zero = jnp.zeros((16,), jnp.float32)
    for r in range(_S // _NS):
        for k8 in range(d // 16):
            zbuf[r, pl.ds(k8 * 16, 16)] = zero
    pltpu.sync_copy(zbuf, acc.at[pl.ds(s * (_S // _NS), _S // _NS)])

    plsc.subcore_barrier()

    def _outer(g, carry):
        for b in range(4):
            i = 4 * g + b
            _repack(i)
            pltpu.make_async_copy(_blk_slice(0), bufs.at[b], sems[b]).wait()

            @pl.when(i + 3 < nb)
            def _refill():
                b2 = (b + 3) % 4
                pltpu.async_copy(_blk_slice(i + 3), bufs.at[b2], sems[b2])

            @pl.when(i < nblk)
            def _scatter():
                pltpu.sync_copy(bufs.at[b], acc.at[ids_v.at[i]], add=True)
        return carry

    lax.fori_loop(0, nb // 4, _outer, 0)

    plsc.subcore_barrier()
    rows = _S // _NS
    pltpu.sync_copy(acc.at[pl.ds(s * rows, rows)],
                    out_hbm.at[pl.ds(c * _S + s * rows, rows)])


def _combine_body(p_ref, o_ref):
    o_ref[...] = p_ref[:_S, :] + p_ref[_S:, :]


def kernel(feat, segment_ids, num_segments):
    del num_segments
    e, d = feat.shape
    total_blocks = e // _BLK
    total_sb = total_blocks // 8
    rem = total_blocks - 8 * total_sb
    qsb = total_sb // _NW
    extra_sb = total_sb - qsb * _NW
    max_blk = max(8 * (qsb + (1 if extra_sb else 0)), 8 * qsb + rem)
    nb = ((max_blk + 3) // 4) * 4

    ids = segment_ids.astype(jnp.int32)

    mesh = plsc.VectorSubcoreMesh(core_axis_name="c", subcore_axis_name="s",
                                  num_cores=_NC, num_subcores=_NS)
    body = functools.partial(_sc_body, total_blocks=total_blocks, qsb=qsb,
                             extra_sb=extra_sb, rem=rem, nb=nb, d=d)
    partials = pl.kernel(
        body,
        jax.ShapeDtypeStruct((_NC * _S, d), jnp.float32),
        mesh=mesh,
        scratch_types=[
            pltpu.VMEM(((nb + 8) * _BLK,), jnp.int32),
            pltpu.VMEM((nb, _BLK), jnp.int32),
            pltpu.VMEM((4, _BLK, d), jnp.float32),
            pltpu.VMEM((_S // _NS, d), jnp.float32),
            pltpu.VMEM_SHARED((_S, d), jnp.float32),
            pltpu.SemaphoreType.DMA,
            pltpu.SemaphoreType.DMA,
            pltpu.SemaphoreType.DMA,
            pltpu.SemaphoreType.DMA,
        ],
    )(feat, ids)

    return pl.pallas_call(
        _combine_body,
        out_shape=jax.ShapeDtypeStruct((_S, d), jnp.float32),
    )(partials)


# speedup vs baseline: 1.2617x; 1.0201x over previous
import functools

import jax
import jax.numpy as jnp
from jax import lax
from jax.experimental import pallas as pl
from jax.experimental.pallas import tpu as pltpu
from jax.experimental.pallas import tpu_sc as plsc

_NC = 2
_NS = 16
_NW = _NC * _NS
_BLK = 128
_S = 512


def _sc_body(feat_hbm, ids_hbm, out_hbm, ids_v, bufs, zbuf, acc,
             sem0, sem1, sem2, sem3,
             *, total_blocks, qsb, extra_sb, rem, nb, d):
    c = lax.axis_index("c")
    s = lax.axis_index("s")
    wid = s * _NC + c
    nblk = 8 * (qsb + (wid < extra_sb).astype(jnp.int32)) \
        + (wid == _NW - 1).astype(jnp.int32) * rem
    start = 8 * (wid * qsb + jnp.minimum(wid, extra_sb))

    def _blk_slice(i):
        bi = jnp.minimum(start + i, total_blocks - 1)
        return feat_hbm.at[pl.ds(bi * _BLK, _BLK)]

    sems = (sem0, sem1, sem2, sem3)
    pltpu.async_copy(_blk_slice(0), bufs.at[0], sems[0])
    pltpu.async_copy(_blk_slice(1), bufs.at[1], sems[1])
    pltpu.async_copy(_blk_slice(2), bufs.at[2], sems[2])

    zero = jnp.zeros((16,), jnp.float32)
    for r in range(_S // _NS):
        for k8 in range(d // 16):
            zbuf[r, pl.ds(k8 * 16, 16)] = zero
    pltpu.sync_copy(zbuf, acc.at[pl.ds(s * (_S // _NS), _S // _NS)])

    pltpu.sync_copy(ids_hbm.at[pl.ds(start, nb)], ids_v)

    plsc.subcore_barrier()

    def _outer(g, carry):
        for b in range(4):
            i = 4 * g + b

            @pl.when(i < nblk)
            def _step():
                pltpu.make_async_copy(_blk_slice(0), bufs.at[b], sems[b]).wait()

                @pl.when(i + 3 < nblk)
                def _refill():
                    b2 = (b + 3) % 4
                    pltpu.async_copy(_blk_slice(i + 3), bufs.at[b2], sems[b2])

                pltpu.sync_copy(bufs.at[b], acc.at[ids_v.at[i]], add=True)
        return carry

    lax.fori_loop(0, nb // 4, _outer, 0)

    plsc.subcore_barrier()
    rows = _S // _NS
    pltpu.sync_copy(acc.at[pl.ds(s * rows, rows)],
                    out_hbm.at[pl.ds(c * _S + s * rows, rows)])


def _combine_body(p_ref, o_ref):
    o_ref[...] = p_ref[:_S, :] + p_ref[_S:, :]


def kernel(feat, segment_ids, num_segments):
    e, d = feat.shape
    total_blocks = e // _BLK
    total_sb = total_blocks // 8
    rem = total_blocks - 8 * total_sb
    qsb = total_sb // _NW
    extra_sb = total_sb - qsb * _NW
    max_blk = max(8 * (qsb + (1 if extra_sb else 0)), 8 * qsb + rem)
    nb = ((max_blk + 3) // 4) * 4

    ids = (segment_ids + (num_segments - _S)).astype(jnp.int32)
    pad_rows = total_blocks + nb
    ids2d = jnp.pad(ids, (0, pad_rows * _BLK - e)).reshape(pad_rows, _BLK)

    mesh = plsc.VectorSubcoreMesh(core_axis_name="c", subcore_axis_name="s",
                                  num_cores=_NC, num_subcores=_NS)
    body = functools.partial(_sc_body, total_blocks=total_blocks, qsb=qsb,
                             extra_sb=extra_sb, rem=rem, nb=nb, d=d)
    partials = pl.kernel(
        body,
        jax.ShapeDtypeStruct((_NC * _S, d), jnp.float32),
        mesh=mesh,
        scratch_types=[
            pltpu.VMEM((nb, _BLK), jnp.int32),
            pltpu.VMEM((4, _BLK, d), jnp.float32),
            pltpu.VMEM((_S // _NS, d), jnp.float32),
            pltpu.VMEM_SHARED((_S, d), jnp.float32),
            pltpu.SemaphoreType.DMA,
            pltpu.SemaphoreType.DMA,
            pltpu.SemaphoreType.DMA,
            pltpu.SemaphoreType.DMA,
        ],
    )(feat, ids2d)

    return pl.pallas_call(
        _combine_body,
        out_shape=jax.ShapeDtypeStruct((_S, d), jnp.float32),
    )(partials)
